# baseline (speedup 1.0000x reference)
import functools

import jax
import jax.numpy as jnp
from jax import lax
from jax.experimental import pallas as pl
from jax.experimental.pallas import tpu as pltpu
from jax.experimental.pallas import tpu_sc as plsc

NW = 32
NS = 16
K = 128


@functools.partial(jax.jit, static_argnames=("n_pad", "epw"))
def _degrees(src, dst, *, n_pad, epw):
    mesh = plsc.VectorSubcoreMesh(core_axis_name="c", subcore_axis_name="s")

    @functools.partial(
        pl.kernel,
        out_type=jax.ShapeDtypeStruct((NW, 2 * n_pad), jnp.float32),
        mesh=mesh,
        scratch_types=[
            pltpu.VMEM((epw,), jnp.int32),
            pltpu.VMEM((epw,), jnp.int32),
            pltpu.VMEM((2 * n_pad,), jnp.float32),
        ],
        compiler_params=pltpu.CompilerParams(needs_layout_passes=False),
    )
    def deg_kernel(src_hbm, dst_hbm, out_hbm, src_v, dst_v, hist_v):
        c = lax.axis_index("c")
        s = lax.axis_index("s")
        w = c * NS + s

        zeros16 = jnp.zeros((16,), jnp.float32)

        def zbody(i, carry):
            hist_v[pl.ds(i * 16, 16)] = zeros16
            return carry

        lax.fori_loop(0, (2 * n_pad) // 16, zbody, 0)

        pltpu.sync_copy(src_hbm.at[pl.ds(w * epw, epw)], src_v)
        pltpu.sync_copy(dst_hbm.at[pl.ds(w * epw, epw)], dst_v)

        ones16 = jnp.ones((16,), jnp.float32)
        offs = jnp.full((16,), n_pad, jnp.int32)

        def body(i, carry):
            sv = src_v[pl.ds(i * 16, 16)]
            dv = dst_v[pl.ds(i * 16, 16)]
            plsc.addupdate_scatter(hist_v, [sv], ones16)
            plsc.addupdate_scatter(hist_v, [dv + offs], ones16)
            return carry

        lax.fori_loop(0, epw // 16, body, 0)
        pltpu.sync_copy(hist_v, out_hbm.at[w])

    return deg_kernel(src, dst)


BR = 1024


@functools.partial(jax.jit, static_argnames=("n", "grid"))
def _scale(deg_src, deg_dst, x, *, n, grid):
    def body(ds_ref, dd_ref, x_ref, y_ref, nd_ref):
        dsrc = jnp.sum(ds_ref[...], axis=0)
        ddst = jnp.sum(dd_ref[...], axis=0)
        ns = lax.rsqrt(jnp.maximum(dsrc, 1.0))
        y_ref[...] = x_ref[...] * ns[:, None]
        nd_ref[...] = lax.rsqrt(jnp.maximum(ddst, 1.0))[None, None, :]

    return pl.pallas_call(
        body,
        grid=(grid,),
        in_specs=[
            pl.BlockSpec((NW, BR), lambda i: (0, i)),
            pl.BlockSpec((NW, BR), lambda i: (0, i)),
            pl.BlockSpec((BR, 128), lambda i: (i, 0)),
        ],
        out_specs=[
            pl.BlockSpec((BR, 128), lambda i: (i, 0)),
            pl.BlockSpec((1, 1, BR), lambda i: (i, 0, 0)),
        ],
        out_shape=[
            jax.ShapeDtypeStruct((n, 128), jnp.float32),
            jax.ShapeDtypeStruct((grid, 1, BR), jnp.float32),
        ],
    )(deg_src, deg_dst, x)


NBUF = 2
G = 16


@functools.partial(jax.jit, static_argnames=("n_pad", "ch"))
def _aggregate(y, src3, dst3, zrows, *, n_pad, ch):
    mesh = plsc.VectorSubcoreMesh(core_axis_name="c", subcore_axis_name="s")
    rpt = n_pad // NS
    groups = ch // G

    @functools.partial(
        pl.kernel,
        out_type=jax.ShapeDtypeStruct((2, n_pad, 128), jnp.float32),
        mesh=mesh,
        scratch_types=[
            pltpu.VMEM((G, K), jnp.int32),
            pltpu.VMEM((G, K), jnp.int32),
            pltpu.VMEM((NBUF, K, 128), jnp.float32),
            pltpu.VMEM_SHARED((n_pad, 128), jnp.float32),
            pltpu.SemaphoreType.DMA,
            pltpu.SemaphoreType.DMA,
        ],
        compiler_params=pltpu.CompilerParams(needs_layout_passes=False),
    )
    def agg_kernel(y_hbm, src_hbm, dst_hbm, z_hbm, out_hbm,
                   sidx_v, didx_v, buf_v, acc_s, gsem, ssem):
        c = lax.axis_index("c")
        s = lax.axis_index("s")
        w = c * NS + s

        pltpu.sync_copy(z_hbm, acc_s.at[pl.ds(s * rpt, rpt)])
        plsc.subcore_barrier()

        def group_body(g, carry):
            pltpu.sync_copy(src_hbm.at[w, pl.ds(g * G, G)], sidx_v)
            pltpu.sync_copy(dst_hbm.at[w, pl.ds(g * G, G)], didx_v)

            for j in range(G):
                slot = j % NBUF
                pltpu.async_copy(
                    y_hbm.at[sidx_v.at[j]], buf_v.at[slot], gsem
                ).wait()
                pltpu.sync_copy(buf_v.at[slot], acc_s.at[didx_v.at[j]],
                                add=True)
            return carry

        lax.fori_loop(0, groups, group_body, 0)

        plsc.subcore_barrier()
        pltpu.sync_copy(acc_s.at[pl.ds(s * rpt, rpt)],
                        out_hbm.at[c, pl.ds(s * rpt, rpt)])

    return agg_kernel(y, src3, dst3, zrows)


@functools.partial(jax.jit, static_argnames=("n", "grid"))
def _finish(accs, nd3, W, b2, *, n, grid):
    def body(acc_ref, nd_ref, w_ref, b_ref, out_ref):
        a = acc_ref[...]
        h = (a[0] + a[1]) * nd_ref[0, 0][:, None]
        out_ref[...] = (
            jnp.dot(h, w_ref[...], preferred_element_type=jnp.float32)
            + b_ref[...]
        )

    return pl.pallas_call(
        body,
        grid=(grid,),
        in_specs=[
            pl.BlockSpec((2, BR, 128), lambda i: (0, i, 0)),
            pl.BlockSpec((1, 1, BR), lambda i: (i, 0, 0)),
            pl.BlockSpec((128, 128), lambda i: (0, 0)),
            pl.BlockSpec((1, 128), lambda i: (0, 0)),
        ],
        out_specs=pl.BlockSpec((BR, 128), lambda i: (i, 0)),
        out_shape=jax.ShapeDtypeStruct((n, 128), jnp.float32),
    )(accs, nd3, W, b2)


def kernel(x, edge_index, W, b):
    n, d_in = x.shape
    e = edge_index.shape[1]
    assert d_in == 128 and W.shape == (128, 128)
    assert e % (NW * 16) == 0

    n_pad = ((n + 1 + 127) // 128) * 128
    ch = (e + NW * K - 1) // (NW * K)
    ch = ((ch + G - 1) // G) * G
    e_pad = NW * ch * K
    grid = (n + BR - 1) // BR

    src = edge_index[0].astype(jnp.int32)
    dst = edge_index[1].astype(jnp.int32)

    deg = _degrees(src, dst, n_pad=n_pad, epw=e // NW)
    deg_src = deg[:, :n_pad]
    deg_dst = deg[:, n_pad:]

    y, nd3 = _scale(deg_src, deg_dst, x, n=n, grid=grid)

    src_p = jnp.concatenate([src, jnp.zeros((e_pad - e,), jnp.int32)])
    dst_p = jnp.concatenate([dst, jnp.full((e_pad - e,), n_pad - 1, jnp.int32)])
    src3 = src_p.reshape(NW, ch, K)
    dst3 = dst_p.reshape(NW, ch, K)
    zrows = jnp.zeros((n_pad // NS, 128), jnp.float32)
    accs = _aggregate(y, src3, dst3, zrows, n_pad=n_pad, ch=ch)

    return _finish(accs, nd3, W, b.reshape(1, 128), n=n, grid=grid)


# speedup vs baseline: 1.5105x; 1.5105x over previous
import functools

import jax
import jax.numpy as jnp
from jax import lax
from jax.experimental import pallas as pl
from jax.experimental.pallas import tpu as pltpu
from jax.experimental.pallas import tpu_sc as plsc

NW = 32
NS = 16
K = 128


@functools.partial(jax.jit, static_argnames=("n_pad", "epw"))
def _degrees(src, dst, *, n_pad, epw):
    mesh = plsc.VectorSubcoreMesh(core_axis_name="c", subcore_axis_name="s")

    @functools.partial(
        pl.kernel,
        out_type=jax.ShapeDtypeStruct((NW, 2 * n_pad), jnp.float32),
        mesh=mesh,
        scratch_types=[
            pltpu.VMEM((epw,), jnp.int32),
            pltpu.VMEM((epw,), jnp.int32),
            pltpu.VMEM((2 * n_pad,), jnp.float32),
        ],
        compiler_params=pltpu.CompilerParams(needs_layout_passes=False),
    )
    def deg_kernel(src_hbm, dst_hbm, out_hbm, src_v, dst_v, hist_v):
        c = lax.axis_index("c")
        s = lax.axis_index("s")
        w = c * NS + s

        zeros16 = jnp.zeros((16,), jnp.float32)

        def zbody(i, carry):
            hist_v[pl.ds(i * 16, 16)] = zeros16
            return carry

        lax.fori_loop(0, (2 * n_pad) // 16, zbody, 0)

        pltpu.sync_copy(src_hbm.at[pl.ds(w * epw, epw)], src_v)
        pltpu.sync_copy(dst_hbm.at[pl.ds(w * epw, epw)], dst_v)

        ones16 = jnp.ones((16,), jnp.float32)
        offs = jnp.full((16,), n_pad, jnp.int32)

        def body(i, carry):
            sv = src_v[pl.ds(i * 16, 16)]
            dv = dst_v[pl.ds(i * 16, 16)]
            plsc.addupdate_scatter(hist_v, [sv], ones16)
            plsc.addupdate_scatter(hist_v, [dv + offs], ones16)
            return carry

        lax.fori_loop(0, epw // 16, body, 0)
        pltpu.sync_copy(hist_v, out_hbm.at[w])

    return deg_kernel(src, dst)


BR = 1024


@functools.partial(jax.jit, static_argnames=("n", "grid"))
def _scale(deg_src, deg_dst, x, *, n, grid):
    def body(ds_ref, dd_ref, x_ref, y_ref, nd_ref):
        dsrc = jnp.sum(ds_ref[...], axis=0)
        ddst = jnp.sum(dd_ref[...], axis=0)
        ns = lax.rsqrt(jnp.maximum(dsrc, 1.0))
        y_ref[...] = x_ref[...] * ns[:, None]
        nd_ref[...] = lax.rsqrt(jnp.maximum(ddst, 1.0))[None, None, :]

    return pl.pallas_call(
        body,
        grid=(grid,),
        in_specs=[
            pl.BlockSpec((NW, BR), lambda i: (0, i)),
            pl.BlockSpec((NW, BR), lambda i: (0, i)),
            pl.BlockSpec((BR, 128), lambda i: (i, 0)),
        ],
        out_specs=[
            pl.BlockSpec((BR, 128), lambda i: (i, 0)),
            pl.BlockSpec((1, 1, BR), lambda i: (i, 0, 0)),
        ],
        out_shape=[
            jax.ShapeDtypeStruct((n, 128), jnp.float32),
            jax.ShapeDtypeStruct((grid, 1, BR), jnp.float32),
        ],
    )(deg_src, deg_dst, x)


@functools.partial(jax.jit, static_argnames=("n_pad", "ch"))
def _aggregate(y, src3, dst3, zrows, *, n_pad, ch):
    mesh = plsc.VectorSubcoreMesh(core_axis_name="c", subcore_axis_name="s")
    rpt = n_pad // NS

    @functools.partial(
        pl.kernel,
        out_type=jax.ShapeDtypeStruct((2, n_pad, 128), jnp.float32),
        mesh=mesh,
        scratch_types=[
            pltpu.VMEM((ch, K), jnp.int32),
            pltpu.VMEM((ch, K), jnp.int32),
            pltpu.VMEM((K, 128), jnp.float32),
            pltpu.VMEM_SHARED((n_pad, 128), jnp.float32),
            pltpu.SemaphoreType.DMA,
        ],
        compiler_params=pltpu.CompilerParams(needs_layout_passes=False),
    )
    def agg_kernel(y_hbm, src_hbm, dst_hbm, z_hbm, out_hbm,
                   sidx_v, didx_v, buf_v, acc_s, gsem):
        c = lax.axis_index("c")
        s = lax.axis_index("s")
        w = c * NS + s

        pltpu.sync_copy(z_hbm, acc_s.at[pl.ds(s * rpt, rpt)])
        pltpu.sync_copy(src_hbm.at[w], sidx_v)
        pltpu.sync_copy(dst_hbm.at[w], didx_v)
        plsc.subcore_barrier()

        def body(j, carry):
            pltpu.async_copy(y_hbm.at[sidx_v.at[j]], buf_v, gsem).wait()
            pltpu.sync_copy(buf_v, acc_s.at[didx_v.at[j]], add=True)
            return carry

        lax.fori_loop(0, ch, body, 0)

        plsc.subcore_barrier()
        pltpu.sync_copy(acc_s.at[pl.ds(s * rpt, rpt)],
                        out_hbm.at[c, pl.ds(s * rpt, rpt)])

    return agg_kernel(y, src3, dst3, zrows)


@functools.partial(jax.jit, static_argnames=("n", "grid"))
def _finish(accs, nd3, W, b2, *, n, grid):
    def body(acc_ref, nd_ref, w_ref, b_ref, out_ref):
        a = acc_ref[...]
        h = (a[0] + a[1]) * nd_ref[0, 0][:, None]
        out_ref[...] = (
            jnp.dot(h, w_ref[...], preferred_element_type=jnp.float32)
            + b_ref[...]
        )

    return pl.pallas_call(
        body,
        grid=(grid,),
        in_specs=[
            pl.BlockSpec((2, BR, 128), lambda i: (0, i, 0)),
            pl.BlockSpec((1, 1, BR), lambda i: (i, 0, 0)),
            pl.BlockSpec((128, 128), lambda i: (0, 0)),
            pl.BlockSpec((1, 128), lambda i: (0, 0)),
        ],
        out_specs=pl.BlockSpec((BR, 128), lambda i: (i, 0)),
        out_shape=jax.ShapeDtypeStruct((n, 128), jnp.float32),
    )(accs, nd3, W, b2)


def kernel(x, edge_index, W, b):
    n, d_in = x.shape
    e = edge_index.shape[1]
    assert d_in == 128 and W.shape == (128, 128)
    assert e % (NW * 16) == 0

    n_pad = ((n + 1 + BR - 1) // BR) * BR
    ch = (e + NW * K - 1) // (NW * K)
    e_pad = NW * ch * K
    grid = (n + BR - 1) // BR

    src = edge_index[0].astype(jnp.int32)
    dst = edge_index[1].astype(jnp.int32)

    deg = _degrees(src, dst, n_pad=n_pad, epw=e // NW)
    deg_src = deg[:, :n_pad]
    deg_dst = deg[:, n_pad:]

    y, nd3 = _scale(deg_src, deg_dst, x, n=n, grid=grid)

    src_p = jnp.concatenate([src, jnp.zeros((e_pad - e,), jnp.int32)])
    dst_p = jnp.concatenate([dst, jnp.full((e_pad - e,), n_pad - 1, jnp.int32)])
    src3 = src_p.reshape(NW, ch, K)
    dst3 = dst_p.reshape(NW, ch, K)
    zrows = jnp.zeros((n_pad // NS, 128), jnp.float32)
    accs = _aggregate(y, src3, dst3, zrows, n_pad=n_pad, ch=ch)

    return _finish(accs, nd3, W, b.reshape(1, 128), n=n, grid=grid)
